# baseline (speedup 1.0000x reference)
import functools

import jax
import jax.numpy as jnp
from jax import lax
from jax.experimental import pallas as pl
from jax.experimental.pallas import tpu as pltpu
from jax.experimental.pallas import tpu_sc as plsc

_NNODES = 10000
_DIM = 256
_ALPHA = 3.0
_K = 20
_N = 4096
_R = 512
_C = 128
_NCHUNK = _N // _C
_HEAD = 16
_NBLK = _N // _R


def _build_sc_gather():
    info = plsc.get_sparse_core_info()
    nc, ns = info.num_cores, info.num_subcores
    nw = nc * ns
    bpw = _N // nw

    mesh = plsc.VectorSubcoreMesh(core_axis_name="c", subcore_axis_name="s")

    @functools.partial(
        pl.kernel,
        mesh=mesh,
        out_type=(
            jax.ShapeDtypeStruct((_N, _DIM), jnp.float32),
            jax.ShapeDtypeStruct((_N, _DIM), jnp.float32),
        ),
        scratch_types=[
            pltpu.VMEM((bpw,), jnp.int32),
            pltpu.VMEM((bpw, _DIM), jnp.float32),
            pltpu.VMEM((bpw, _DIM), jnp.float32),
            pltpu.SemaphoreType.DMA,
            pltpu.SemaphoreType.DMA,
        ],
    )
    def gather_k(idx_hbm, t1_hbm, t2_hbm, o1_hbm, o2_hbm, idx_v, r1, r2, s1, s2):
        wid = lax.axis_index("s") * nc + lax.axis_index("c")
        base = wid * bpw
        pltpu.sync_copy(idx_hbm.at[pl.ds(base, bpw)], idx_v)
        c1 = pltpu.async_copy(t1_hbm.at[idx_v], r1, s1)
        c2 = pltpu.async_copy(t2_hbm.at[idx_v], r2, s2)
        c1.wait()
        c2.wait()
        pltpu.sync_copy(r1, o1_hbm.at[pl.ds(base, bpw)])
        pltpu.sync_copy(r2, o2_hbm.at[pl.ds(base, bpw)])

    return gather_k


_sc_gather = None


def _gather(idx, emb1, emb2):
    global _sc_gather
    if _sc_gather is None:
        _sc_gather = _build_sc_gather()
    return _sc_gather(idx, emb1, emb2)


def _nv_compute(g1_ref, g2_ref, w1_ref, b1_ref, w2_ref, b2_ref, nv1_ref, nv2_ref):
    m1 = lax.dot_general(g1_ref[...], w1_ref[...], (((1,), (1,)), ((), ())))
    nv1_ref[...] = jnp.tanh(_ALPHA * (m1 + b1_ref[...]))
    m2 = lax.dot_general(g2_ref[...], w2_ref[...], (((1,), (1,)), ((), ())))
    nv2_ref[...] = jnp.tanh(_ALPHA * (m2 + b2_ref[...]))


def _adj_block(i, nv1_ref, nv2_ref):
    nv1b = nv1_ref[pl.ds(i * _R, _R), :]
    nv2b = nv2_ref[pl.ds(i * _R, _R), :]
    a = lax.dot_general(nv1b, nv2_ref[...], (((1,), (1,)), ((), ())))
    a = a - lax.dot_general(nv2b, nv1_ref[...], (((1,), (1,)), ((), ())))
    return jnp.maximum(jnp.tanh(_ALPHA * a), 0.0)


def _main_body(g1_ref, g2_ref, w1_ref, b1_ref, w2_ref, b2_ref, tmat_ref,
               out_ref, flag_ref, nv1_ref, nv2_ref):
    i = pl.program_id(0)

    @pl.when(i == 0)
    def _():
        _nv_compute(g1_ref, g2_ref, w1_ref, b1_ref, w2_ref, b2_ref,
                    nv1_ref, nv2_ref)

    adj = _adj_block(i, nv1_ref, nv2_ref)

    kf = jnp.float32(_K)
    c1 = jnp.sum(jnp.where(adj >= 1.0, 1.0, 0.0), axis=1, keepdims=True)
    tm = tmat_ref[...]

    base = jnp.zeros((_R, 1), jnp.float32)
    for c in range(_HEAD):
        adjc = adj[:, c * _C:(c + 1) * _C]
        eq = adjc == 1.0
        eqf = jnp.where(eq, 1.0, 0.0)
        p = lax.dot_general(eqf.astype(jnp.bfloat16), tm,
                            (((1,), (0,)), ((), ())),
                            preferred_element_type=jnp.float32)
        sel = eq & ((p + base) < kf)
        out_ref[:, c * _C:(c + 1) * _C] = jnp.where(sel, 1.0, 0.0)
        base = base + jnp.sum(eqf, axis=1, keepdims=True)

    zeros_tail = jnp.zeros((_R, _C), jnp.float32)
    for c in range(_HEAD, _NCHUNK):
        out_ref[:, c * _C:(c + 1) * _C] = zeros_tail

    bad = jnp.any(base < kf)
    flag_ref[...] = jnp.where(bad, 1.0, 0.0) * jnp.ones((1, 1, _C), jnp.float32)


def _main_call(g1, g2, W1, b1, W2, b2, tmat):
    return pl.pallas_call(
        _main_body,
        grid=(_NBLK,),
        in_specs=[
            pl.BlockSpec((_N, _DIM), lambda i: (0, 0)),
            pl.BlockSpec((_N, _DIM), lambda i: (0, 0)),
            pl.BlockSpec((_DIM, _DIM), lambda i: (0, 0)),
            pl.BlockSpec((1, _DIM), lambda i: (0, 0)),
            pl.BlockSpec((_DIM, _DIM), lambda i: (0, 0)),
            pl.BlockSpec((1, _DIM), lambda i: (0, 0)),
            pl.BlockSpec((_C, _C), lambda i: (0, 0)),
        ],
        out_specs=(
            pl.BlockSpec((_R, _N), lambda i: (i, 0)),
            pl.BlockSpec((1, 1, _C), lambda i: (i, 0, 0)),
        ),
        out_shape=(
            jax.ShapeDtypeStruct((_N, _N), jnp.float32),
            jax.ShapeDtypeStruct((_NBLK, 1, _C), jnp.float32),
        ),
        scratch_shapes=[
            pltpu.VMEM((_N, _DIM), jnp.float32),
            pltpu.VMEM((_N, _DIM), jnp.float32),
        ],
    )(g1, g2, W1, b1.reshape(1, _DIM), W2, b2.reshape(1, _DIM), tmat)


def _fix_body(main_out_ref, flags_ref, g1_any, g2_any, w1_ref, b1_ref,
              w2_ref, b2_ref, tmat_ref, out_ref,
              g1v, g2v, nv1_ref, nv2_ref, blk_ref,
              t_ref, tf_ref, gt_ref, cnt_ref, sem):
    i = pl.program_id(0)
    anybad = jnp.any(flags_ref[...] > 0.0)

    @pl.when(jnp.logical_and(i == 0, anybad))
    def _():
        cp1 = pltpu.make_async_copy(g1_any, g1v, sem)
        cp1.start()
        cp1.wait()
        cp2 = pltpu.make_async_copy(g2_any, g2v, sem)
        cp2.start()
        cp2.wait()
        _nv_compute(g1v, g2v, w1_ref, b1_ref, w2_ref, b2_ref,
                    nv1_ref, nv2_ref)

    blkbad = jnp.any(flags_ref[pl.ds(i, 1), 0, :] > 0.0)

    @pl.when(blkbad)
    def _():
        adj = _adj_block(i, nv1_ref, nv2_ref)
        blk_ref[...] = adj
        kf = jnp.float32(_K)
        zeros = jnp.zeros((_R, 1), jnp.float32)
        t_ref[...] = jnp.max(adj, axis=1, keepdims=True)
        tf_ref[...] = zeros
        gt_ref[...] = zeros
        cnt_ref[...] = zeros

        def cond(done):
            return jnp.logical_not(done)

        def body(done):
            adjv = blk_ref[...]
            t = t_ref[...]
            eq = adjv == t
            c = jnp.sum(eq.astype(jnp.float32), axis=1, keepdims=True)
            m2 = jnp.max(jnp.where(adjv < t, adjv, -1.0), axis=1,
                         keepdims=True)
            cnt = cnt_ref[...]
            active = cnt < kf
            gt_ref[...] = jnp.where(active, cnt, gt_ref[...])
            tf_ref[...] = jnp.where(active, t, tf_ref[...])
            newcnt = jnp.where(active, cnt + c, cnt)
            cnt_ref[...] = newcnt
            t_ref[...] = jnp.where(active, m2, t)
            return jnp.all(newcnt >= kf)

        lax.while_loop(cond, body, jnp.bool_(False))

        t = tf_ref[...]
        need = kf - gt_ref[...]
        tm = tmat_ref[...]
        base = zeros
        for c in range(_NCHUNK):
            adjc = blk_ref[:, c * _C:(c + 1) * _C]
            eq = adjc == t
            eqf = jnp.where(eq, 1.0, 0.0)
            p = lax.dot_general(eqf.astype(jnp.bfloat16), tm,
                                (((1,), (0,)), ((), ())),
                                preferred_element_type=jnp.float32)
            sel = (adjc > t) | (eq & ((p + base) < need))
            blk_ref[:, c * _C:(c + 1) * _C] = jnp.where(sel, adjc, 0.0)
            base = base + jnp.sum(eqf, axis=1, keepdims=True)

        cp = pltpu.make_async_copy(
            blk_ref, out_ref.at[pl.ds(i * _R, _R), :], sem)
        cp.start()
        cp.wait()


def _fix_call(main_out, flags, g1, g2, W1, b1, W2, b2, tmat):
    return pl.pallas_call(
        _fix_body,
        grid=(_NBLK,),
        in_specs=[
            pl.BlockSpec(memory_space=pl.ANY),
            pl.BlockSpec((_NBLK, 1, _C), lambda i: (0, 0, 0)),
            pl.BlockSpec(memory_space=pl.ANY),
            pl.BlockSpec(memory_space=pl.ANY),
            pl.BlockSpec((_DIM, _DIM), lambda i: (0, 0)),
            pl.BlockSpec((1, _DIM), lambda i: (0, 0)),
            pl.BlockSpec((_DIM, _DIM), lambda i: (0, 0)),
            pl.BlockSpec((1, _DIM), lambda i: (0, 0)),
            pl.BlockSpec((_C, _C), lambda i: (0, 0)),
        ],
        out_specs=pl.BlockSpec(memory_space=pl.ANY),
        out_shape=jax.ShapeDtypeStruct((_N, _N), jnp.float32),
        input_output_aliases={0: 0},
        scratch_shapes=[
            pltpu.VMEM((_N, _DIM), jnp.float32),
            pltpu.VMEM((_N, _DIM), jnp.float32),
            pltpu.VMEM((_N, _DIM), jnp.float32),
            pltpu.VMEM((_N, _DIM), jnp.float32),
            pltpu.VMEM((_R, _N), jnp.float32),
            pltpu.VMEM((_R, 1), jnp.float32),
            pltpu.VMEM((_R, 1), jnp.float32),
            pltpu.VMEM((_R, 1), jnp.float32),
            pltpu.VMEM((_R, 1), jnp.float32),
            pltpu.SemaphoreType.DMA,
        ],
    )(main_out, flags, g1, g2, W1, b1.reshape(1, _DIM),
      W2, b2.reshape(1, _DIM), tmat)


def _make_tmat():
    l = jnp.arange(_C)[:, None]
    j = jnp.arange(_C)[None, :]
    return jnp.where(l < j, 1.0, 0.0).astype(jnp.bfloat16)


def kernel(idx, emb1, emb2, W1, b1, W2, b2):
    g1, g2 = _gather(idx.astype(jnp.int32), emb1, emb2)
    tmat = _make_tmat()
    out, flags = _main_call(g1, g2, W1, b1, W2, b2, tmat)
    return _fix_call(out, flags, g1, g2, W1, b1, W2, b2, tmat)


# speedup vs baseline: 1.0230x; 1.0230x over previous
import functools

import jax
import jax.numpy as jnp
from jax import lax
from jax.experimental import pallas as pl
from jax.experimental.pallas import tpu as pltpu
from jax.experimental.pallas import tpu_sc as plsc

_NNODES = 10000
_DIM = 256
_ALPHA = 3.0
_K = 20
_N = 4096
_R = 512
_C = 128
_NCHUNK = _N // _C
_HEAD = 16
_NBLK = _N // _R


def _build_sc_gather():
    info = plsc.get_sparse_core_info()
    nc, ns = info.num_cores, info.num_subcores
    nw = nc * ns
    bpw = _N // nw

    mesh = plsc.VectorSubcoreMesh(core_axis_name="c", subcore_axis_name="s")

    @functools.partial(
        pl.kernel,
        mesh=mesh,
        out_type=(
            jax.ShapeDtypeStruct((_N, _DIM), jnp.float32),
            jax.ShapeDtypeStruct((_N, _DIM), jnp.float32),
        ),
        scratch_types=[
            pltpu.VMEM((bpw,), jnp.int32),
            pltpu.VMEM((bpw, _DIM), jnp.float32),
            pltpu.VMEM((bpw, _DIM), jnp.float32),
            pltpu.SemaphoreType.DMA,
            pltpu.SemaphoreType.DMA,
        ],
    )
    def gather_k(idx_hbm, t1_hbm, t2_hbm, o1_hbm, o2_hbm, idx_v, r1, r2, s1, s2):
        wid = lax.axis_index("s") * nc + lax.axis_index("c")
        base = wid * bpw
        pltpu.sync_copy(idx_hbm.at[pl.ds(base, bpw)], idx_v)
        c1 = pltpu.async_copy(t1_hbm.at[idx_v], r1, s1)
        c2 = pltpu.async_copy(t2_hbm.at[idx_v], r2, s2)
        c1.wait()
        c2.wait()
        pltpu.sync_copy(r1, o1_hbm.at[pl.ds(base, bpw)])
        pltpu.sync_copy(r2, o2_hbm.at[pl.ds(base, bpw)])

    return gather_k


_sc_gather = None


def _gather(idx, emb1, emb2):
    global _sc_gather
    if _sc_gather is None:
        _sc_gather = _build_sc_gather()
    return _sc_gather(idx, emb1, emb2)


def _nv_compute(g1_ref, g2_ref, w1_ref, b1_ref, w2_ref, b2_ref, nv1_ref, nv2_ref):
    m1 = lax.dot_general(g1_ref[...], w1_ref[...], (((1,), (1,)), ((), ())))
    nv1_ref[...] = jnp.tanh(_ALPHA * (m1 + b1_ref[...]))
    m2 = lax.dot_general(g2_ref[...], w2_ref[...], (((1,), (1,)), ((), ())))
    nv2_ref[...] = jnp.tanh(_ALPHA * (m2 + b2_ref[...]))


def _adj_block(i, nv1_ref, nv2_ref):
    nv1b = nv1_ref[pl.ds(i * _R, _R), :]
    nv2b = nv2_ref[pl.ds(i * _R, _R), :]
    a = lax.dot_general(nv1b, nv2_ref[...], (((1,), (1,)), ((), ())))
    a = a - lax.dot_general(nv2b, nv1_ref[...], (((1,), (1,)), ((), ())))
    return jnp.maximum(jnp.tanh(_ALPHA * a), 0.0)


def _main_body(g1_ref, g2_ref, w1_ref, b1_ref, w2_ref, b2_ref, tmat_ref,
               out_ref, flag_ref, nv1_ref, nv2_ref):
    i = pl.program_id(0)

    @pl.when(i == 0)
    def _():
        _nv_compute(g1_ref, g2_ref, w1_ref, b1_ref, w2_ref, b2_ref,
                    nv1_ref, nv2_ref)

    adj = _adj_block(i, nv1_ref, nv2_ref)

    kf = jnp.float32(_K)
    tm = tmat_ref[...]

    base = jnp.zeros((_R, 1), jnp.float32)
    for c in range(_HEAD):
        adjc = adj[:, c * _C:(c + 1) * _C]
        eq = adjc == 1.0
        eqf = jnp.where(eq, 1.0, 0.0)
        p = lax.dot_general(eqf.astype(jnp.bfloat16), tm,
                            (((1,), (0,)), ((), ())),
                            preferred_element_type=jnp.float32)
        sel = eq & ((p + base) < kf)
        out_ref[:, c * _C:(c + 1) * _C] = jnp.where(sel, 1.0, 0.0)
        base = base + jnp.sum(eqf, axis=1, keepdims=True)

    out_ref[:, _HEAD * _C:] = jnp.zeros((_R, _N - _HEAD * _C), jnp.float32)

    bad = jnp.any(base < kf)
    flag_ref[...] = jnp.where(bad, 1.0, 0.0) * jnp.ones((1, 1, _C), jnp.float32)


def _main_call(g1, g2, W1, b1, W2, b2, tmat):
    return pl.pallas_call(
        _main_body,
        grid=(_NBLK,),
        in_specs=[
            pl.BlockSpec((_N, _DIM), lambda i: (0, 0)),
            pl.BlockSpec((_N, _DIM), lambda i: (0, 0)),
            pl.BlockSpec((_DIM, _DIM), lambda i: (0, 0)),
            pl.BlockSpec((1, _DIM), lambda i: (0, 0)),
            pl.BlockSpec((_DIM, _DIM), lambda i: (0, 0)),
            pl.BlockSpec((1, _DIM), lambda i: (0, 0)),
            pl.BlockSpec((_C, _C), lambda i: (0, 0)),
        ],
        out_specs=(
            pl.BlockSpec((_R, _N), lambda i: (i, 0)),
            pl.BlockSpec((1, 1, _C), lambda i: (i, 0, 0)),
        ),
        out_shape=(
            jax.ShapeDtypeStruct((_N, _N), jnp.float32),
            jax.ShapeDtypeStruct((_NBLK, 1, _C), jnp.float32),
        ),
        scratch_shapes=[
            pltpu.VMEM((_N, _DIM), jnp.float32),
            pltpu.VMEM((_N, _DIM), jnp.float32),
        ],
    )(g1, g2, W1, b1.reshape(1, _DIM), W2, b2.reshape(1, _DIM), tmat)


def _fix_body(main_out_ref, flags_ref, g1_any, g2_any, w1_ref, b1_ref,
              w2_ref, b2_ref, tmat_ref, out_ref,
              g1v, g2v, nv1_ref, nv2_ref, blk_ref,
              t_ref, tf_ref, gt_ref, cnt_ref, sem):
    i = pl.program_id(0)
    anybad = jnp.any(flags_ref[...] > 0.0)

    @pl.when(jnp.logical_and(i == 0, anybad))
    def _():
        cp1 = pltpu.make_async_copy(g1_any, g1v, sem)
        cp1.start()
        cp1.wait()
        cp2 = pltpu.make_async_copy(g2_any, g2v, sem)
        cp2.start()
        cp2.wait()
        _nv_compute(g1v, g2v, w1_ref, b1_ref, w2_ref, b2_ref,
                    nv1_ref, nv2_ref)

    blkbad = jnp.any(flags_ref[pl.ds(i, 1), 0, :] > 0.0)

    @pl.when(blkbad)
    def _():
        adj = _adj_block(i, nv1_ref, nv2_ref)
        blk_ref[...] = adj
        kf = jnp.float32(_K)
        zeros = jnp.zeros((_R, 1), jnp.float32)
        t_ref[...] = jnp.max(adj, axis=1, keepdims=True)
        tf_ref[...] = zeros
        gt_ref[...] = zeros
        cnt_ref[...] = zeros

        def cond(done):
            return jnp.logical_not(done)

        def body(done):
            adjv = blk_ref[...]
            t = t_ref[...]
            eq = adjv == t
            c = jnp.sum(eq.astype(jnp.float32), axis=1, keepdims=True)
            m2 = jnp.max(jnp.where(adjv < t, adjv, -1.0), axis=1,
                         keepdims=True)
            cnt = cnt_ref[...]
            active = cnt < kf
            gt_ref[...] = jnp.where(active, cnt, gt_ref[...])
            tf_ref[...] = jnp.where(active, t, tf_ref[...])
            newcnt = jnp.where(active, cnt + c, cnt)
            cnt_ref[...] = newcnt
            t_ref[...] = jnp.where(active, m2, t)
            return jnp.all(newcnt >= kf)

        lax.while_loop(cond, body, jnp.bool_(False))

        t = tf_ref[...]
        need = kf - gt_ref[...]
        tm = tmat_ref[...]
        base = zeros
        for c in range(_NCHUNK):
            adjc = blk_ref[:, c * _C:(c + 1) * _C]
            eq = adjc == t
            eqf = jnp.where(eq, 1.0, 0.0)
            p = lax.dot_general(eqf.astype(jnp.bfloat16), tm,
                                (((1,), (0,)), ((), ())),
                                preferred_element_type=jnp.float32)
            sel = (adjc > t) | (eq & ((p + base) < need))
            blk_ref[:, c * _C:(c + 1) * _C] = jnp.where(sel, adjc, 0.0)
            base = base + jnp.sum(eqf, axis=1, keepdims=True)

        cp = pltpu.make_async_copy(
            blk_ref, out_ref.at[pl.ds(i * _R, _R), :], sem)
        cp.start()
        cp.wait()


def _fix_call(main_out, flags, g1, g2, W1, b1, W2, b2, tmat):
    return pl.pallas_call(
        _fix_body,
        grid=(_NBLK,),
        in_specs=[
            pl.BlockSpec(memory_space=pl.ANY),
            pl.BlockSpec((_NBLK, 1, _C), lambda i: (0, 0, 0)),
            pl.BlockSpec(memory_space=pl.ANY),
            pl.BlockSpec(memory_space=pl.ANY),
            pl.BlockSpec((_DIM, _DIM), lambda i: (0, 0)),
            pl.BlockSpec((1, _DIM), lambda i: (0, 0)),
            pl.BlockSpec((_DIM, _DIM), lambda i: (0, 0)),
            pl.BlockSpec((1, _DIM), lambda i: (0, 0)),
            pl.BlockSpec((_C, _C), lambda i: (0, 0)),
        ],
        out_specs=pl.BlockSpec(memory_space=pl.ANY),
        out_shape=jax.ShapeDtypeStruct((_N, _N), jnp.float32),
        input_output_aliases={0: 0},
        scratch_shapes=[
            pltpu.VMEM((_N, _DIM), jnp.float32),
            pltpu.VMEM((_N, _DIM), jnp.float32),
            pltpu.VMEM((_N, _DIM), jnp.float32),
            pltpu.VMEM((_N, _DIM), jnp.float32),
            pltpu.VMEM((_R, _N), jnp.float32),
            pltpu.VMEM((_R, 1), jnp.float32),
            pltpu.VMEM((_R, 1), jnp.float32),
            pltpu.VMEM((_R, 1), jnp.float32),
            pltpu.VMEM((_R, 1), jnp.float32),
            pltpu.SemaphoreType.DMA,
        ],
    )(main_out, flags, g1, g2, W1, b1.reshape(1, _DIM),
      W2, b2.reshape(1, _DIM), tmat)


def _make_tmat():
    l = jnp.arange(_C)[:, None]
    j = jnp.arange(_C)[None, :]
    return jnp.where(l < j, 1.0, 0.0).astype(jnp.bfloat16)


def kernel(idx, emb1, emb2, W1, b1, W2, b2):
    g1, g2 = _gather(idx.astype(jnp.int32), emb1, emb2)
    tmat = _make_tmat()
    out, flags = _main_call(g1, g2, W1, b1, W2, b2, tmat)
    return _fix_call(out, flags, g1, g2, W1, b1, W2, b2, tmat)
